# baseline (speedup 1.0000x reference)
import functools
import math

import jax
import jax.numpy as jnp
from jax import lax
from jax.experimental import pallas as pl
from jax.experimental.pallas import tpu as pltpu
from jax.experimental.pallas import tpu_sc as plsc

VOCAB = 1000000
EMBED_DIM = 64
SCALE = 8.0

NUM_CORES = 2
NUM_SUBCORES = 16
NW = NUM_CORES * NUM_SUBCORES

B_I = 4096
B_J = 200
NT_I = B_I // 128
NBLK = B_J * NT_I
BLK_PER_W = NBLK // NW
DEPTH = 2


def _make_kernel():
    mesh = plsc.VectorSubcoreMesh(core_axis_name="c", subcore_axis_name="s")

    @functools.partial(
        pl.kernel,
        mesh=mesh,
        out_type=jax.ShapeDtypeStruct((B_J, 8, NT_I, 8, 128), jnp.float32),
        scratch_types=[
            pltpu.VMEM((BLK_PER_W, 128), jnp.int32),
            pltpu.VMEM((DEPTH, 128), jnp.int32),
            pltpu.VMEM((DEPTH, 128, EMBED_DIM), jnp.float32),
            pltpu.VMEM((2, EMBED_DIM, 128), jnp.float32),
            pltpu.SemaphoreType.DMA((DEPTH,)),
            pltpu.SemaphoreType.DMA((2,)),
        ],
        compiler_params=pltpu.CompilerParams(
            use_tc_tiling_on_sc=False, needs_layout_passes=False
        ),
    )
    def k(xt_hbm, tp_hbm, out_hbm, vbuf, rbuf, gbuf, obuf, sem_g, sem_o):
        wid = lax.axis_index("s") * NUM_CORES + lax.axis_index("c")
        base_blk = wid * BLK_PER_W
        iota16 = lax.iota(jnp.int32, 16)

        pltpu.sync_copy(xt_hbm.at[pl.ds(base_blk, BLK_PER_W)], vbuf)

        def prep_and_fire(local_b, s):
            for g in range(8):
                v16 = vbuf[local_b, pl.ds(g * 16, 16)]
                rbuf[s, pl.ds(g * 16, 16)] = lax.shift_left(v16, 1)
            pltpu.async_copy(tp_hbm.at[rbuf.at[s]], gbuf.at[s], sem_g.at[s])

        def do_block(b, s, p):
            blk = base_blk + b
            j = blk // NT_I
            it = lax.rem(blk, NT_I)

            @pl.when(b + DEPTH - 1 < BLK_PER_W)
            def _():
                prep_and_fire(b + DEPTH - 1, (s + DEPTH - 1) % DEPTH)

            pltpu.make_async_copy(tp_hbm.at[rbuf.at[s]], gbuf.at[s],
                                  sem_g.at[s]).wait()

            @pl.when(b >= 2)
            def _():
                blk2 = blk - 2
                j2 = blk2 // NT_I
                it2 = lax.rem(blk2, NT_I)
                for a in range(8):
                    pltpu.make_async_copy(
                        obuf.at[p, pl.ds(a * 8, 8)],
                        out_hbm.at[j2, a, it2], sem_o.at[p]).wait()

            gslot = gbuf.at[s]
            for g in range(8):
                rows = iota16 + (g * 16)
                sl = pl.ds(g * 16, 16)
                for d0 in range(0, EMBED_DIM, 8):
                    vals = [
                        plsc.load_gather(gslot, [rows, iota16 * 0 + (d0 + t)])
                        for t in range(8)
                    ]
                    for t in range(8):
                        obuf[p, d0 + t, sl] = vals[t] * SCALE

            for a in range(8):
                pltpu.async_copy(obuf.at[p, pl.ds(a * 8, 8)],
                                 out_hbm.at[j, a, it], sem_o.at[p])

        for s in range(DEPTH - 1):
            prep_and_fire(s, s)

        def pair_body(b2, carry):
            for u in range(2):
                do_block(2 * b2 + u, u, u)
            return carry

        lax.fori_loop(0, BLK_PER_W // 2, pair_body, 0)

        for tail in (BLK_PER_W - 2, BLK_PER_W - 1):
            blk = base_blk + tail
            j = blk // NT_I
            it = lax.rem(blk, NT_I)
            p = tail % 2
            for a in range(8):
                pltpu.make_async_copy(obuf.at[p, pl.ds(a * 8, 8)],
                                      out_hbm.at[j, a, it],
                                      sem_o.at[p]).wait()

    return k


def kernel(x, table):
    tp = jnp.pad(table, ((0, 0), (0, EMBED_DIM))).reshape(2 * VOCAB,
                                                          EMBED_DIM)
    xt2 = x.T.reshape(NBLK, 128).astype(jnp.int32)
    out5 = _make_kernel()(xt2, tp)
    return out5.transpose(2, 4, 0, 1, 3).reshape(B_I, B_J, EMBED_DIM)


# speedup vs baseline: 1.0135x; 1.0135x over previous
import functools
import math

import jax
import jax.numpy as jnp
from jax import lax
from jax.experimental import pallas as pl
from jax.experimental.pallas import tpu as pltpu
from jax.experimental.pallas import tpu_sc as plsc

VOCAB = 1000000
EMBED_DIM = 64
SCALE = 8.0

NUM_CORES = 2
NUM_SUBCORES = 16
NW = NUM_CORES * NUM_SUBCORES

B_I = 4096
B_J = 200
NT_I = B_I // 128
NBLK = B_J * NT_I
BLK_PER_W = NBLK // NW
DEPTH = 2


def _make_kernel():
    mesh = plsc.VectorSubcoreMesh(core_axis_name="c", subcore_axis_name="s")

    @functools.partial(
        pl.kernel,
        mesh=mesh,
        out_type=jax.ShapeDtypeStruct((B_J, 8, NT_I, 8, 128), jnp.float32),
        scratch_types=[
            pltpu.VMEM((BLK_PER_W, 128), jnp.int32),
            pltpu.VMEM((DEPTH, 128), jnp.int32),
            pltpu.VMEM((DEPTH, 128, EMBED_DIM), jnp.float32),
            pltpu.VMEM((2, EMBED_DIM, 129), jnp.float32),
            pltpu.SemaphoreType.DMA((DEPTH,)),
            pltpu.SemaphoreType.DMA((2,)),
        ],
        compiler_params=pltpu.CompilerParams(
            use_tc_tiling_on_sc=False, needs_layout_passes=False
        ),
    )
    def k(xt_hbm, tp_hbm, out_hbm, vbuf, rbuf, gbuf, obuf, sem_g, sem_o):
        wid = lax.axis_index("s") * NUM_CORES + lax.axis_index("c")
        base_blk = wid * BLK_PER_W
        iota16 = lax.iota(jnp.int32, 16)

        pltpu.sync_copy(xt_hbm.at[pl.ds(base_blk, BLK_PER_W)], vbuf)

        def prep_and_fire(local_b, s):
            for g in range(8):
                v16 = vbuf[local_b, pl.ds(g * 16, 16)]
                rbuf[s, pl.ds(g * 16, 16)] = lax.shift_left(v16, 1)
            pltpu.async_copy(tp_hbm.at[rbuf.at[s]], gbuf.at[s], sem_g.at[s])

        def do_block(b, s, p):
            blk = base_blk + b
            j = blk // NT_I
            it = lax.rem(blk, NT_I)

            @pl.when(b + DEPTH - 1 < BLK_PER_W)
            def _():
                prep_and_fire(b + DEPTH - 1, (s + DEPTH - 1) % DEPTH)

            pltpu.make_async_copy(tp_hbm.at[rbuf.at[s]], gbuf.at[s],
                                  sem_g.at[s]).wait()

            @pl.when(b >= 2)
            def _():
                blk2 = blk - 2
                j2 = blk2 // NT_I
                it2 = lax.rem(blk2, NT_I)
                for a in range(8):
                    pltpu.make_async_copy(
                        obuf.at[p, pl.ds(a * 8, 8), pl.ds(0, 128)],
                        out_hbm.at[j2, a, it2], sem_o.at[p]).wait()

            gslot = gbuf.at[s]
            oslot = obuf.at[p]
            for il in range(128):
                for d0 in range(0, EMBED_DIM, 16):
                    vals = gslot[il, pl.ds(d0, 16)] * SCALE
                    plsc.store_scatter(oslot, [iota16 + d0, iota16 * 0 + il],
                                       vals)

            for a in range(8):
                pltpu.async_copy(obuf.at[p, pl.ds(a * 8, 8), pl.ds(0, 128)],
                                 out_hbm.at[j, a, it], sem_o.at[p])

        for s in range(DEPTH - 1):
            prep_and_fire(s, s)

        def pair_body(b2, carry):
            for u in range(2):
                do_block(2 * b2 + u, u, u)
            return carry

        lax.fori_loop(0, BLK_PER_W // 2, pair_body, 0)

        for tail in (BLK_PER_W - 2, BLK_PER_W - 1):
            blk = base_blk + tail
            j = blk // NT_I
            it = lax.rem(blk, NT_I)
            p = tail % 2
            for a in range(8):
                pltpu.make_async_copy(obuf.at[p, pl.ds(a * 8, 8),
                                              pl.ds(0, 128)],
                                      out_hbm.at[j, a, it],
                                      sem_o.at[p]).wait()

    return k


def kernel(x, table):
    tp = jnp.pad(table, ((0, 0), (0, EMBED_DIM))).reshape(2 * VOCAB,
                                                          EMBED_DIM)
    xt2 = x.T.reshape(NBLK, 128).astype(jnp.int32)
    out5 = _make_kernel()(xt2, tp)
    return out5.transpose(2, 4, 0, 1, 3).reshape(B_I, B_J, EMBED_DIM)


# speedup vs baseline: 1.2611x; 1.2444x over previous
import functools
import math

import jax
import jax.numpy as jnp
from jax import lax
from jax.experimental import pallas as pl
from jax.experimental.pallas import tpu as pltpu
from jax.experimental.pallas import tpu_sc as plsc

VOCAB = 1000000
EMBED_DIM = 64
SCALE = 8.0

NUM_CORES = 2
NUM_SUBCORES = 16
NW = NUM_CORES * NUM_SUBCORES

B_I = 4096
B_J = 200
NT_I = B_I // 128
NBLK = B_J * NT_I
BLK_PER_W = NBLK // NW
DEPTH = 2


def _make_kernel():
    mesh = plsc.VectorSubcoreMesh(core_axis_name="c", subcore_axis_name="s")

    @functools.partial(
        pl.kernel,
        mesh=mesh,
        out_type=jax.ShapeDtypeStruct((B_J, 8, NT_I, 8, 128), jnp.float32),
        scratch_types=[
            pltpu.VMEM((BLK_PER_W, 128), jnp.int32),
            pltpu.VMEM((DEPTH, 128), jnp.int32),
            pltpu.VMEM((DEPTH, 128, EMBED_DIM), jnp.float32),
            pltpu.VMEM((2, EMBED_DIM, 129), jnp.float32),
            pltpu.SemaphoreType.DMA((DEPTH,)),
            pltpu.SemaphoreType.DMA((2,)),
        ],
        compiler_params=pltpu.CompilerParams(
            use_tc_tiling_on_sc=False, needs_layout_passes=False
        ),
    )
    def k(xt_hbm, tp_hbm, out_hbm, vbuf, rbuf, gbuf, obuf, sem_g, sem_o):
        wid = lax.axis_index("s") * NUM_CORES + lax.axis_index("c")
        base_blk = wid * BLK_PER_W
        iota16 = lax.iota(jnp.int32, 16)

        pltpu.sync_copy(xt_hbm.at[pl.ds(base_blk, BLK_PER_W)], vbuf)

        def prep_and_fire(local_b, s):
            for g in range(8):
                v16 = vbuf[local_b, pl.ds(g * 16, 16)]
                rbuf[s, pl.ds(g * 16, 16)] = lax.shift_left(v16, 1)
            pltpu.async_copy(tp_hbm.at[rbuf.at[s]], gbuf.at[s], sem_g.at[s])

        def do_block(b, s, p):
            blk = base_blk + b
            j = blk // NT_I
            it = lax.rem(blk, NT_I)

            @pl.when(b + DEPTH - 1 < BLK_PER_W)
            def _():
                prep_and_fire(b + DEPTH - 1, (s + DEPTH - 1) % DEPTH)

            pltpu.make_async_copy(tp_hbm.at[rbuf.at[s]], gbuf.at[s],
                                  sem_g.at[s]).wait()

            @pl.when(b >= 2)
            def _():
                blk2 = blk - 2
                j2 = blk2 // NT_I
                it2 = lax.rem(blk2, NT_I)
                for a in range(8):
                    pltpu.make_async_copy(
                        obuf.at[p, pl.ds(a * 8, 8), pl.ds(0, 128)],
                        out_hbm.at[j2, a, it2], sem_o.at[p]).wait()

            gslot = gbuf.at[s]
            oslot = obuf.at[p]

            def il_body(il4, carry):
                for u in range(4):
                    il = il4 * 4 + u
                    cols = iota16 * 0 + il
                    for d0 in range(0, EMBED_DIM, 16):
                        vals = gslot[il, pl.ds(d0, 16)] * SCALE
                        plsc.store_scatter(oslot, [iota16 + d0, cols], vals)
                return carry

            lax.fori_loop(0, 32, il_body, 0)

            for a in range(8):
                pltpu.async_copy(obuf.at[p, pl.ds(a * 8, 8), pl.ds(0, 128)],
                                 out_hbm.at[j, a, it], sem_o.at[p])

        for s in range(DEPTH - 1):
            prep_and_fire(s, s)

        def pair_body(b2, carry):
            for u in range(2):
                do_block(2 * b2 + u, u, u)
            return carry

        lax.fori_loop(0, BLK_PER_W // 2, pair_body, 0)

        for tail in (BLK_PER_W - 2, BLK_PER_W - 1):
            blk = base_blk + tail
            j = blk // NT_I
            it = lax.rem(blk, NT_I)
            p = tail % 2
            for a in range(8):
                pltpu.make_async_copy(obuf.at[p, pl.ds(a * 8, 8),
                                              pl.ds(0, 128)],
                                      out_hbm.at[j, a, it],
                                      sem_o.at[p]).wait()

    return k


def kernel(x, table):
    tp = jnp.pad(table, ((0, 0), (0, EMBED_DIM))).reshape(2 * VOCAB,
                                                          EMBED_DIM)
    xt2 = x.T.reshape(NBLK, 128).astype(jnp.int32)
    out5 = _make_kernel()(xt2, tp)
    return out5.transpose(2, 4, 0, 1, 3).reshape(B_I, B_J, EMBED_DIM)


# speedup vs baseline: 1.8248x; 1.4470x over previous
import functools
import math

import jax
import jax.numpy as jnp
from jax import lax
from jax.experimental import pallas as pl
from jax.experimental.pallas import tpu as pltpu
from jax.experimental.pallas import tpu_sc as plsc

VOCAB = 1000000
EMBED_DIM = 64
SCALE = 8.0

NUM_CORES = 2
NUM_SUBCORES = 16
NW = NUM_CORES * NUM_SUBCORES

B_I = 4096
B_J = 200
NT_I = B_I // 128
NBLK = B_J * NT_I
BLK_PER_W = NBLK // NW
DEPTH = 2


def _make_kernel():
    mesh = plsc.VectorSubcoreMesh(core_axis_name="c", subcore_axis_name="s")

    @functools.partial(
        pl.kernel,
        mesh=mesh,
        out_type=jax.ShapeDtypeStruct((B_J, 8, NT_I, 8, 128), jnp.float32),
        scratch_types=[
            pltpu.VMEM((BLK_PER_W, 128), jnp.int32),
            pltpu.VMEM((DEPTH, 128), jnp.int32),
            pltpu.VMEM((DEPTH, 128, EMBED_DIM), jnp.float32),
            pltpu.VMEM((2, EMBED_DIM, 129), jnp.float32),
            pltpu.SemaphoreType.DMA((DEPTH,)),
            pltpu.SemaphoreType.DMA((2,)),
        ],
        compiler_params=pltpu.CompilerParams(
            use_tc_tiling_on_sc=False, needs_layout_passes=False
        ),
    )
    def k(xt_hbm, tp_hbm, out_hbm, vbuf, rbuf, gbuf, obuf, sem_g, sem_o):
        wid = lax.axis_index("s") * NUM_CORES + lax.axis_index("c")
        base_blk = wid * BLK_PER_W
        iota16 = lax.iota(jnp.int32, 16)

        pltpu.sync_copy(xt_hbm.at[pl.ds(base_blk, BLK_PER_W)], vbuf)

        def prep_and_fire(local_b, s):
            for g in range(8):
                v16 = vbuf[local_b, pl.ds(g * 16, 16)]
                rbuf[s, pl.ds(g * 16, 16)] = lax.shift_left(v16, 1)
            pltpu.async_copy(tp_hbm.at[rbuf.at[s]], gbuf.at[s], sem_g.at[s])

        def do_block(b, s, p):
            blk = base_blk + b
            j = blk // NT_I
            it = lax.rem(blk, NT_I)

            @pl.when(b + DEPTH - 1 < BLK_PER_W)
            def _():
                prep_and_fire(b + DEPTH - 1, (s + DEPTH - 1) % DEPTH)

            pltpu.make_async_copy(tp_hbm.at[rbuf.at[s]], gbuf.at[s],
                                  sem_g.at[s]).wait()

            @pl.when(b >= 2)
            def _():
                blk2 = blk - 2
                j2 = blk2 // NT_I
                it2 = lax.rem(blk2, NT_I)
                for a in range(8):
                    pltpu.make_async_copy(
                        obuf.at[p, pl.ds(a * 8, 8), pl.ds(0, 128)],
                        out_hbm.at[j2, a, it2], sem_o.at[p]).wait()

            gslot = gbuf.at[s]
            oslot = obuf.at[p]

            @plsc.parallel_loop(0, 128, unroll=4)
            def _(il):
                cols = iota16 * 0 + il
                for d0 in range(0, EMBED_DIM, 16):
                    vals = gslot[il, pl.ds(d0, 16)] * SCALE
                    plsc.store_scatter(oslot, [iota16 + d0, cols], vals)

            for a in range(8):
                pltpu.async_copy(obuf.at[p, pl.ds(a * 8, 8), pl.ds(0, 128)],
                                 out_hbm.at[j, a, it], sem_o.at[p])

        for s in range(DEPTH - 1):
            prep_and_fire(s, s)

        def pair_body(b2, carry):
            for u in range(2):
                do_block(2 * b2 + u, u, u)
            return carry

        lax.fori_loop(0, BLK_PER_W // 2, pair_body, 0)

        for tail in (BLK_PER_W - 2, BLK_PER_W - 1):
            blk = base_blk + tail
            j = blk // NT_I
            it = lax.rem(blk, NT_I)
            p = tail % 2
            for a in range(8):
                pltpu.make_async_copy(obuf.at[p, pl.ds(a * 8, 8),
                                              pl.ds(0, 128)],
                                      out_hbm.at[j, a, it],
                                      sem_o.at[p]).wait()

    return k


def kernel(x, table):
    tp = jnp.pad(table, ((0, 0), (0, EMBED_DIM))).reshape(2 * VOCAB,
                                                          EMBED_DIM)
    xt2 = x.T.reshape(NBLK, 128).astype(jnp.int32)
    out5 = _make_kernel()(xt2, tp)
    return out5.transpose(2, 4, 0, 1, 3).reshape(B_I, B_J, EMBED_DIM)


# speedup vs baseline: 1.9013x; 1.0419x over previous
import functools
import math

import jax
import jax.numpy as jnp
from jax import lax
from jax.experimental import pallas as pl
from jax.experimental.pallas import tpu as pltpu
from jax.experimental.pallas import tpu_sc as plsc

VOCAB = 1000000
EMBED_DIM = 64
SCALE = 8.0

NUM_CORES = 2
NUM_SUBCORES = 16
NW = NUM_CORES * NUM_SUBCORES

B_I = 4096
B_J = 200
NT_I = B_I // 128
NBLK = B_J * NT_I
SB_PER_W = NBLK // NW // 2
DEPTH = 2


def _make_kernel():
    mesh = plsc.VectorSubcoreMesh(core_axis_name="c", subcore_axis_name="s")

    @functools.partial(
        pl.kernel,
        mesh=mesh,
        out_type=jax.ShapeDtypeStruct((B_J, 8, NT_I, 8, 128), jnp.float32),
        scratch_types=[
            pltpu.VMEM((SB_PER_W, 256), jnp.int32),
            pltpu.VMEM((DEPTH, 2, 128), jnp.int32),
            pltpu.VMEM((DEPTH, 256, EMBED_DIM), jnp.float32),
            pltpu.VMEM((2, 2, EMBED_DIM, 129), jnp.float32),
            pltpu.SemaphoreType.DMA((DEPTH,)),
            pltpu.SemaphoreType.DMA((2,)),
        ],
        compiler_params=pltpu.CompilerParams(
            use_tc_tiling_on_sc=False, needs_layout_passes=False
        ),
    )
    def k(xt_hbm, tp_hbm, out_hbm, vbuf, rbuf, gbuf, obuf, sem_g, sem_o):
        wid = lax.axis_index("s") * NUM_CORES + lax.axis_index("c")
        base_sb = wid * SB_PER_W
        iota16 = lax.iota(jnp.int32, 16)

        pltpu.sync_copy(xt_hbm.at[pl.ds(base_sb, SB_PER_W)], vbuf)

        def prep_and_fire(local_sb, s):
            for h in range(2):
                for g in range(8):
                    v16 = vbuf[local_sb, pl.ds(h * 128 + g * 16, 16)]
                    rbuf[s, h, pl.ds(g * 16, 16)] = lax.shift_left(v16, 1)
            for h in range(2):
                pltpu.async_copy(tp_hbm.at[rbuf.at[s, h]],
                                 gbuf.at[s, pl.ds(h * 128, 128)],
                                 sem_g.at[s])

        def do_sblock(sb, s):
            blk0 = (base_sb + sb) * 2
            j0 = blk0 // NT_I
            it0 = lax.rem(blk0, NT_I)
            blk1 = blk0 + 1
            j1 = blk1 // NT_I
            it1 = lax.rem(blk1, NT_I)

            @pl.when(sb + DEPTH - 1 < SB_PER_W)
            def _():
                prep_and_fire(sb + DEPTH - 1, (s + DEPTH - 1) % DEPTH)

            for h in range(2):
                pltpu.make_async_copy(tp_hbm.at[rbuf.at[s, h]],
                                      gbuf.at[s, pl.ds(h * 128, 128)],
                                      sem_g.at[s]).wait()

            @pl.when(sb >= 2)
            def _():
                blk2 = blk0 - 4
                j2 = blk2 // NT_I
                it2 = lax.rem(blk2, NT_I)
                blk3 = blk2 + 1
                j3 = blk3 // NT_I
                it3 = lax.rem(blk3, NT_I)
                for a in range(8):
                    pltpu.make_async_copy(
                        obuf.at[s, 0, pl.ds(a * 8, 8), pl.ds(0, 128)],
                        out_hbm.at[j2, a, it2], sem_o.at[s]).wait()
                for a in range(8):
                    pltpu.make_async_copy(
                        obuf.at[s, 1, pl.ds(a * 8, 8), pl.ds(0, 128)],
                        out_hbm.at[j3, a, it3], sem_o.at[s]).wait()

            for h in range(2):
                gslot = gbuf.at[s, pl.ds(h * 128, 128)]
                oslot = obuf.at[s, h]

                @plsc.parallel_loop(0, 128, unroll=4)
                def _(il):
                    cols = iota16 * 0 + il
                    for d0 in range(0, EMBED_DIM, 16):
                        vals = gslot[il, pl.ds(d0, 16)] * SCALE
                        plsc.store_scatter(oslot, [iota16 + d0, cols], vals)

            for a in range(8):
                pltpu.async_copy(
                    obuf.at[s, 0, pl.ds(a * 8, 8), pl.ds(0, 128)],
                    out_hbm.at[j0, a, it0], sem_o.at[s])
            for a in range(8):
                pltpu.async_copy(
                    obuf.at[s, 1, pl.ds(a * 8, 8), pl.ds(0, 128)],
                    out_hbm.at[j1, a, it1], sem_o.at[s])

        for s in range(DEPTH - 1):
            prep_and_fire(s, s)

        def pair_body(sb2, carry):
            for u in range(2):
                do_sblock(2 * sb2 + u, u)
            return carry

        lax.fori_loop(0, SB_PER_W // 2, pair_body, 0)

        for tail in (SB_PER_W - 2, SB_PER_W - 1):
            s = tail % 2
            for half in range(2):
                blk = (base_sb + tail) * 2 + half
                j = blk // NT_I
                it = lax.rem(blk, NT_I)
                for a in range(8):
                    pltpu.make_async_copy(
                        obuf.at[s, half, pl.ds(a * 8, 8), pl.ds(0, 128)],
                        out_hbm.at[j, a, it], sem_o.at[s]).wait()

    return k


def kernel(x, table):
    tp = jnp.pad(table, ((0, 0), (0, EMBED_DIM))).reshape(2 * VOCAB,
                                                          EMBED_DIM)
    xt3 = x.T.reshape(NBLK // 2, 256).astype(jnp.int32)
    out5 = _make_kernel()(xt3, tp)
    return out5.transpose(2, 4, 0, 1, 3).reshape(B_I, B_J, EMBED_DIM)


# speedup vs baseline: 1.9045x; 1.0017x over previous
import functools
import math

import jax
import jax.numpy as jnp
from jax import lax
from jax.experimental import pallas as pl
from jax.experimental.pallas import tpu as pltpu
from jax.experimental.pallas import tpu_sc as plsc

VOCAB = 1000000
EMBED_DIM = 64
SCALE = 8.0

NUM_CORES = 2
NUM_SUBCORES = 16
NW = NUM_CORES * NUM_SUBCORES

B_I = 4096
B_J = 200
NT_I = B_I // 128
NBLK = B_J * NT_I
SB_PER_W = NBLK // NW // 2
DEPTH = 4


def _make_kernel():
    mesh = plsc.VectorSubcoreMesh(core_axis_name="c", subcore_axis_name="s")

    @functools.partial(
        pl.kernel,
        mesh=mesh,
        out_type=jax.ShapeDtypeStruct((B_J, 8, NT_I, 8, 128), jnp.float32),
        scratch_types=[
            pltpu.VMEM((SB_PER_W, 256), jnp.int32),
            pltpu.VMEM((DEPTH, 2, 128), jnp.int32),
            pltpu.VMEM((DEPTH, 256, EMBED_DIM), jnp.float32),
            pltpu.VMEM((2, 2, EMBED_DIM, 129), jnp.float32),
            pltpu.SemaphoreType.DMA((DEPTH,)),
            pltpu.SemaphoreType.DMA((2,)),
        ],
        compiler_params=pltpu.CompilerParams(
            use_tc_tiling_on_sc=False, needs_layout_passes=False
        ),
    )
    def k(xt_hbm, tp_hbm, out_hbm, vbuf, rbuf, gbuf, obuf, sem_g, sem_o):
        wid = lax.axis_index("s") * NUM_CORES + lax.axis_index("c")
        base_sb = wid * SB_PER_W
        iota16 = lax.iota(jnp.int32, 16)

        pltpu.sync_copy(xt_hbm.at[pl.ds(base_sb, SB_PER_W)], vbuf)

        def prep_and_fire(local_sb, s):
            for h in range(2):
                for g in range(8):
                    v16 = vbuf[local_sb, pl.ds(h * 128 + g * 16, 16)]
                    rbuf[s, h, pl.ds(g * 16, 16)] = lax.shift_left(v16, 1)
            for h in range(2):
                pltpu.async_copy(tp_hbm.at[rbuf.at[s, h]],
                                 gbuf.at[s, pl.ds(h * 128, 128)],
                                 sem_g.at[s])

        def do_sblock(sb, s, p):
            blk0 = (base_sb + sb) * 2
            j0 = blk0 // NT_I
            it0 = lax.rem(blk0, NT_I)
            blk1 = blk0 + 1
            j1 = blk1 // NT_I
            it1 = lax.rem(blk1, NT_I)

            @pl.when(sb + DEPTH - 1 < SB_PER_W)
            def _():
                prep_and_fire(sb + DEPTH - 1, (s + DEPTH - 1) % DEPTH)

            for h in range(2):
                pltpu.make_async_copy(tp_hbm.at[rbuf.at[s, h]],
                                      gbuf.at[s, pl.ds(h * 128, 128)],
                                      sem_g.at[s]).wait()

            @pl.when(sb >= 2)
            def _():
                blk2 = blk0 - 4
                j2 = blk2 // NT_I
                it2 = lax.rem(blk2, NT_I)
                blk3 = blk2 + 1
                j3 = blk3 // NT_I
                it3 = lax.rem(blk3, NT_I)
                for a in range(8):
                    pltpu.make_async_copy(
                        obuf.at[p, 0, pl.ds(a * 8, 8), pl.ds(0, 128)],
                        out_hbm.at[j2, a, it2], sem_o.at[p]).wait()
                for a in range(8):
                    pltpu.make_async_copy(
                        obuf.at[p, 1, pl.ds(a * 8, 8), pl.ds(0, 128)],
                        out_hbm.at[j3, a, it3], sem_o.at[p]).wait()

            for h in range(2):
                gslot = gbuf.at[s, pl.ds(h * 128, 128)]
                oslot = obuf.at[p, h]

                @plsc.parallel_loop(0, 128, unroll=4)
                def _(il):
                    cols = iota16 * 0 + il
                    for d0 in range(0, EMBED_DIM, 16):
                        vals = gslot[il, pl.ds(d0, 16)] * SCALE
                        plsc.store_scatter(oslot, [iota16 + d0, cols], vals)

            for a in range(8):
                pltpu.async_copy(
                    obuf.at[p, 0, pl.ds(a * 8, 8), pl.ds(0, 128)],
                    out_hbm.at[j0, a, it0], sem_o.at[p])
            for a in range(8):
                pltpu.async_copy(
                    obuf.at[p, 1, pl.ds(a * 8, 8), pl.ds(0, 128)],
                    out_hbm.at[j1, a, it1], sem_o.at[p])

        for s in range(DEPTH - 1):
            prep_and_fire(s, s)

        def quad_body(sb4, carry):
            for u in range(DEPTH):
                do_sblock(DEPTH * sb4 + u, u, u % 2)
            return carry

        lax.fori_loop(0, SB_PER_W // DEPTH, quad_body, 0)

        for tail in (SB_PER_W - 2, SB_PER_W - 1):
            s = tail % 2
            for half in range(2):
                blk = (base_sb + tail) * 2 + half
                j = blk // NT_I
                it = lax.rem(blk, NT_I)
                for a in range(8):
                    pltpu.make_async_copy(
                        obuf.at[s, half, pl.ds(a * 8, 8), pl.ds(0, 128)],
                        out_hbm.at[j, a, it], sem_o.at[s]).wait()

    return k


def kernel(x, table):
    tp = jnp.pad(table, ((0, 0), (0, EMBED_DIM))).reshape(2 * VOCAB,
                                                          EMBED_DIM)
    xt3 = x.T.reshape(NBLK // 2, 256).astype(jnp.int32)
    out5 = _make_kernel()(xt3, tp)
    return out5.transpose(2, 4, 0, 1, 3).reshape(B_I, B_J, EMBED_DIM)
